# Initial kernel scaffold; baseline (speedup 1.0000x reference)
import jax
import jax.numpy as jnp
from jax.experimental import pallas as pl


def kernel(x, edge_index):
    raise NotImplementedError("write your pallas kernel here")


# speedup vs baseline: 10.8206x; 10.8206x over previous
import functools

import jax
import jax.numpy as jnp
from jax import lax
from jax.experimental import pallas as pl
from jax.experimental.pallas import tpu as pltpu
from jax.experimental.pallas import tpu_sc as plsc

_N = 10000
_E = 320000
_D = 128
_K = 4
_LAM = 3.0
_NACC = 10112
_PAD = _NACC - _N
_NC = 2
_NS = 16
_NW = _NC * _NS
_EPW = _E // _NW
_CB = 80
_NCH = _EPW // _CB
_RPS = _NACC // _NS
_ZR = 64
_DEG_CH = 2000


def _lane_sum(v):
    for sh in (1, 2, 4, 8):
        idx = lax.iota(jnp.int32, 16) ^ sh
        v = v + v.at[idx].get(mode="promise_in_bounds")
    return v


def _rsqrt_nr(x):
    i = plsc.bitcast(x, jnp.int32)
    y = plsc.bitcast(jnp.full((16,), 0x5F3759DF, jnp.int32) - (i >> 1), jnp.float32)
    for _ in range(3):
        y = y * (1.5 - 0.5 * x * y * y)
    return y


def _zero_buf(ref, rows):
    z16 = jnp.zeros((16,), jnp.float32)

    def body(i, _):
        for j in range(_D // 16):
            ref[i, pl.ds(j * 16, 16)] = z16
        return 0

    lax.fori_loop(0, rows, body, 0)


def _init_acc(acc, zb_v, s):
    base = s * _RPS
    nfull = _RPS // _ZR
    for k in range(nfull):
        pltpu.sync_copy(zb_v, acc.at[pl.ds(base + k * _ZR, _ZR)])
    rem = _RPS - nfull * _ZR
    if rem:
        pltpu.sync_copy(zb_v.at[pl.ds(0, rem)], acc.at[pl.ds(base + nfull * _ZR, rem)])


def _write_acc(acc, out_hbm, c, s):
    rb = s * _RPS
    pltpu.sync_copy(acc.at[pl.ds(rb, _RPS)], out_hbm.at[c, pl.ds(rb, _RPS)])


def _deg_body(row_hbm, col_hbm, out_hbm, rv, cv, dp):
    c = lax.axis_index("c")
    s = lax.axis_index("s")
    wid = s * _NC + c
    z16 = jnp.zeros((16,), jnp.float32)

    def zb(i, _):
        dp[pl.ds(i * 16, 16)] = z16
        return 0

    lax.fori_loop(0, _NACC // 16, zb, 0)

    ones = jnp.full((16,), 1.0, jnp.float32)
    for kk in range(_EPW // _DEG_CH):
        off = pl.multiple_of(wid * _EPW + kk * _DEG_CH, 8)
        pltpu.sync_copy(row_hbm.at[pl.ds(off, _DEG_CH)], rv)
        pltpu.sync_copy(col_hbm.at[pl.ds(off, _DEG_CH)], cv)

        def inner(i, _):
            r = rv[pl.ds(i * 16, 16)]
            cc = cv[pl.ds(i * 16, 16)]
            plsc.addupdate_scatter(dp, [r], ones, mask=r != cc)
            return 0

        lax.fori_loop(0, _DEG_CH // 16, inner, 0)
    pltpu.sync_copy(dp, out_hbm.at[wid])


@functools.cache
def _deg_call():
    mesh = plsc.VectorSubcoreMesh(core_axis_name="c", subcore_axis_name="s")
    return pl.kernel(
        _deg_body,
        out_type=jax.ShapeDtypeStruct((_NW, _NACC), jnp.float32),
        mesh=mesh,
        scratch_types=[
            pltpu.VMEM((_DEG_CH,), jnp.int32),
            pltpu.VMEM((_DEG_CH,), jnp.int32),
            pltpu.VMEM((_NACC,), jnp.float32),
        ],
        compiler_params=pltpu.CompilerParams(needs_layout_passes=False),
    )


def _adj_body(up_hbm, colx_hbm, adst_hbm, out_hbm, gi_v, si_v, rows_v, zb_v, acc, sem):
    c = lax.axis_index("c")
    s = lax.axis_index("s")
    wid = s * _NC + c
    _zero_buf(zb_v, _ZR)
    _init_acc(acc, zb_v, s)
    plsc.subcore_barrier()
    eb = wid * _EPW

    def chunk(k, _):
        off = pl.multiple_of(eb + k * _CB, 8)
        pltpu.sync_copy(colx_hbm.at[pl.ds(off, _CB)], gi_v)
        pltpu.sync_copy(adst_hbm.at[pl.ds(off, _CB)], si_v)
        pltpu.async_copy(up_hbm.at[gi_v], rows_v, sem).wait()
        pltpu.sync_copy(rows_v, acc.at[si_v], add=True)
        return 0

    lax.fori_loop(0, _NCH, chunk, 0)
    plsc.subcore_barrier()
    _write_acc(acc, out_hbm, c, s)


@functools.cache
def _adj_call():
    mesh = plsc.VectorSubcoreMesh(core_axis_name="c", subcore_axis_name="s")
    return pl.kernel(
        _adj_body,
        out_type=jax.ShapeDtypeStruct((_NC, _NACC, _D), jnp.float32),
        mesh=mesh,
        scratch_types=[
            pltpu.VMEM((_CB,), jnp.int32),
            pltpu.VMEM((_CB,), jnp.int32),
            pltpu.VMEM((_CB, _D), jnp.float32),
            pltpu.VMEM((_ZR, _D), jnp.float32),
            pltpu.VMEM_SHARED((_NACC, _D), jnp.float32),
            pltpu.SemaphoreType.DMA,
        ],
        compiler_params=pltpu.CompilerParams(needs_layout_passes=False),
    )


def _zupd_body(wp_hbm, arow_hbm, acol_hbm, zin_hbm, zout_hbm, pt_hbm,
               ia_v, ib_v, wr_v, wc_v, zv_v, zn_v, zb_v, acc, sem1, sem2):
    c = lax.axis_index("c")
    s = lax.axis_index("s")
    wid = s * _NC + c
    _zero_buf(zb_v, _ZR)
    _init_acc(acc, zb_v, s)
    plsc.subcore_barrier()
    eb = wid * _EPW
    one = jnp.full((16,), 1.0, jnp.float32)

    def chunk(k, _):
        off = pl.multiple_of(eb + k * _CB, 8)
        pltpu.sync_copy(arow_hbm.at[pl.ds(off, _CB)], ia_v)
        pltpu.sync_copy(acol_hbm.at[pl.ds(off, _CB)], ib_v)
        cp1 = pltpu.async_copy(wp_hbm.at[ia_v], wr_v, sem1)
        cp2 = pltpu.async_copy(wp_hbm.at[ib_v], wc_v, sem2)
        pltpu.sync_copy(zin_hbm.at[pl.ds(off, _CB)], zv_v)
        cp1.wait()
        cp2.wait()

        def rows(r4, _2):
            for u in range(2):
                r = r4 * 2 + u
                zb = []
                ss = None
                for j in range(_D // 16):
                    sl = pl.ds(j * 16, 16)
                    b = zv_v[r, sl] + 2.0 * (wr_v[r, sl] - wc_v[r, sl])
                    zb.append(b)
                    ss = b * b if ss is None else ss + b * b
                tot = _lane_sum(ss)
                scale = jnp.minimum(one, _LAM * _rsqrt_nr(tot))
                for j in range(_D // 16):
                    sl = pl.ds(j * 16, 16)
                    v = scale * zb[j]
                    zv_v[r, sl] = v
                    zn_v[r, sl] = -v
            return 0

        lax.fori_loop(0, _CB // 2, rows, 0)
        pltpu.sync_copy(zv_v, zout_hbm.at[pl.ds(off, _CB)])
        pltpu.sync_copy(zv_v, acc.at[ia_v], add=True)
        pltpu.sync_copy(zn_v, acc.at[ib_v], add=True)
        return 0

    lax.fori_loop(0, _NCH, chunk, 0)
    plsc.subcore_barrier()
    _write_acc(acc, pt_hbm, c, s)


@functools.cache
def _zupd_call():
    mesh = plsc.VectorSubcoreMesh(core_axis_name="c", subcore_axis_name="s")
    return pl.kernel(
        _zupd_body,
        out_type=(
            jax.ShapeDtypeStruct((_E, _D), jnp.float32),
            jax.ShapeDtypeStruct((_NC, _NACC, _D), jnp.float32),
        ),
        mesh=mesh,
        scratch_types=[
            pltpu.VMEM((_CB,), jnp.int32),
            pltpu.VMEM((_CB,), jnp.int32),
            pltpu.VMEM((_CB, _D), jnp.float32),
            pltpu.VMEM((_CB, _D), jnp.float32),
            pltpu.VMEM((_CB, _D), jnp.float32),
            pltpu.VMEM((_CB, _D), jnp.float32),
            pltpu.VMEM((_ZR, _D), jnp.float32),
            pltpu.VMEM_SHARED((_NACC, _D), jnp.float32),
            pltpu.SemaphoreType.DMA,
            pltpu.SemaphoreType.DMA,
        ],
        compiler_params=pltpu.CompilerParams(needs_layout_passes=False),
    )


_BR = 1264
_G = _NACC // _BR


def _tc_b(xp, xcp, t, p0, p1, dis_col):
    def body(xp_r, xc_r, t_r, p0_r, p1_r, dis_r, y_r, w_r):
        dis = dis_r[...]
        adj = dis * (p0_r[...] + p1_r[...]) + (dis * dis) * xc_r[...]
        y = 0.25 * xp_r[...] + 0.75 * adj
        y_r[...] = y
        w_r[...] = dis * (y - 0.25 * t_r[...])

    bs = pl.BlockSpec((_BR, _D), lambda i: (i, 0))
    bs1 = pl.BlockSpec((_BR, 1), lambda i: (i, 0))
    return pl.pallas_call(
        body,
        grid=(_G,),
        in_specs=[bs, bs, bs, bs, bs, bs1],
        out_specs=[bs, bs],
        out_shape=[jax.ShapeDtypeStruct((_NACC, _D), jnp.float32)] * 2,
    )(xp, xcp, t, p0, p1, dis_col)


def _tc_d(y, s0, s1, dis_col):
    def body(y_r, s0_r, s1_r, dis_r, t_r, xc_r, up_r):
        dis = dis_r[...]
        t = dis * (s0_r[...] + s1_r[...])
        xc = y_r[...] - 0.25 * t
        t_r[...] = t
        xc_r[...] = xc
        up_r[...] = dis * xc

    bs = pl.BlockSpec((_BR, _D), lambda i: (i, 0))
    bs1 = pl.BlockSpec((_BR, 1), lambda i: (i, 0))
    return pl.pallas_call(
        body,
        grid=(_G,),
        in_specs=[bs, bs, bs, bs1],
        out_specs=[bs, bs, bs],
        out_shape=[jax.ShapeDtypeStruct((_NACC, _D), jnp.float32)] * 3,
    )(y, s0, s1, dis_col)


def kernel(x, edge_index):
    row = edge_index[0].astype(jnp.int32)
    col = edge_index[1].astype(jnp.int32)

    e_ar = jnp.arange(_E, dtype=jnp.int32)
    spread1 = _N + (e_ar % _PAD)
    spread2 = _N + ((e_ar + _PAD // 2) % _PAD)
    nd = row != col
    adst = jnp.where(nd, row, spread1)
    act = row > col
    arow = jnp.where(act, row, spread1)
    acol = jnp.where(act, col, spread2)

    degp = _deg_call()(row, col)
    deg = 1.0 + jnp.sum(degp, axis=0)
    ii = jnp.arange(_NACC)
    dis = jnp.where(ii < _N, lax.rsqrt(deg), 0.0).astype(jnp.float32)
    dis_col = dis[:, None]

    xp = jnp.zeros((_NACC, _D), jnp.float32).at[:_N].set(x)
    zacc = jnp.zeros((_NACC, _D), jnp.float32)
    t, xcp, up = _tc_d(xp, zacc, zacc, dis_col)
    z = jnp.zeros((_E, _D), jnp.float32)

    for _ in range(_K):
        p = _adj_call()(up, col, adst)
        y, w = _tc_b(xp, xcp, t, p[0], p[1], dis_col)
        z, sp = _zupd_call()(w, arow, acol, z)
        t, xcp, up = _tc_d(y, sp[0], sp[1], dis_col)
    return xcp[:_N]


# speedup vs baseline: 22.3545x; 2.0659x over previous
import functools

import jax
import jax.numpy as jnp
from jax import lax
from jax.experimental import pallas as pl
from jax.experimental.pallas import tpu as pltpu
from jax.experimental.pallas import tpu_sc as plsc

_N = 10000
_E = 320000
_D = 128
_K = 4
_LAM = 3.0
_NACC = 10016
_PAD = _NACC - _N
_NC = 2
_NS = 16
_NW = _NC * _NS
_EPW = _E // _NW
_CBA = 80
_NCHA = _EPW // _CBA
_CBZ = 40
_RPS = _NACC // _NS
_ZR = 16
_DEG_CH = 2000


def _lane_sum(v):
    for sh in (1, 2, 4, 8):
        idx = lax.iota(jnp.int32, 16) ^ sh
        v = v + v.at[idx].get(mode="promise_in_bounds")
    return v


def _rsqrt_nr(x):
    i = plsc.bitcast(x, jnp.int32)
    y = plsc.bitcast(jnp.full((16,), 0x5F3759DF, jnp.int32) - (i >> 1), jnp.float32)
    for _ in range(3):
        y = y * (1.5 - 0.5 * x * y * y)
    return y


def _zero_buf(ref, rows):
    z16 = jnp.zeros((16,), jnp.float32)

    def body(i, _):
        for j in range(_D // 16):
            ref[i, pl.ds(j * 16, 16)] = z16
        return 0

    lax.fori_loop(0, rows, body, 0)


_RPT = 624


def _init_acc(acc, zb_v, s):
    base = s * _RPT

    def cp(k, _):
        pltpu.sync_copy(zb_v, acc.at[pl.ds(base + k * _ZR, _ZR)])
        return 0

    lax.fori_loop(0, _RPT // _ZR, cp, 0)

    @pl.when(s == _NS - 1)
    def _():
        pltpu.sync_copy(zb_v, acc.at[pl.ds(_NS * _RPT, _ZR)])
        pltpu.sync_copy(zb_v, acc.at[pl.ds(_NS * _RPT + _ZR, _ZR)])


def _write_acc(acc, out_hbm, c, s):
    rb = pl.multiple_of(s * _RPT, 8)
    pltpu.sync_copy(acc.at[pl.ds(rb, _RPT)], out_hbm.at[c, pl.ds(rb, _RPT)])

    @pl.when(s == _NS - 1)
    def _():
        tail = _NS * _RPT
        pltpu.sync_copy(
            acc.at[pl.ds(tail, _NACC - tail)],
            out_hbm.at[c, pl.ds(tail, _NACC - tail)],
        )


def _deg_body(row_hbm, col_hbm, degp_hbm, arc_hbm, acc_hbm, cnt_hbm,
              rv, cv, dp, arl, acl, cb16):
    c = lax.axis_index("c")
    s = lax.axis_index("s")
    wid = s * _NC + c
    z16 = jnp.zeros((16,), jnp.float32)

    def zb(i, _):
        dp[pl.ds(i * 16, 16)] = z16
        return 0

    lax.fori_loop(0, _NACC // 16, zb, 0)

    ones = jnp.full((16,), 1.0, jnp.float32)
    eb = wid * _EPW
    cnt = jnp.int32(0)
    for kk in range(_EPW // _DEG_CH):
        off = pl.multiple_of(eb + kk * _DEG_CH, 8)
        pltpu.sync_copy(row_hbm.at[pl.ds(off, _DEG_CH)], rv)
        pltpu.sync_copy(col_hbm.at[pl.ds(off, _DEG_CH)], cv)

        def inner(i, cnt_c):
            r = rv[pl.ds(i * 16, 16)]
            cc = cv[pl.ds(i * 16, 16)]
            plsc.addupdate_scatter(dp, [r], ones, mask=r != cc)
            m_act = r > cc
            plsc.store_compressed(arl.at[pl.ds(cnt_c, 16)], r, mask=m_act)
            plsc.store_compressed(acl.at[pl.ds(cnt_c, 16)], cc, mask=m_act)
            pc = plsc.all_reduce_population_count(m_act)
            return cnt_c + jnp.max(pc)

        cnt = lax.fori_loop(0, _DEG_CH // 16, inner, cnt)

    base_i = lax.iota(jnp.int32, 16)
    for j in range(3):
        dv1 = _N + ((cnt + j * 16 + base_i + wid) % _PAD)
        dv2 = _N + ((cnt + j * 16 + base_i + 7 * wid + 13) % _PAD)
        arl[pl.ds(cnt + j * 16, 16)] = dv1
        acl[pl.ds(cnt + j * 16, 16)] = dv2

    nch = jnp.maximum((cnt + _CBZ - 1) // _CBZ, 1)
    cb16[...] = jnp.broadcast_to(nch, (16,)).astype(jnp.int32)
    pltpu.sync_copy(cb16, cnt_hbm.at[wid])
    pltpu.sync_copy(arl.at[pl.ds(0, _EPW)], arc_hbm.at[pl.ds(eb, _EPW)])
    pltpu.sync_copy(acl.at[pl.ds(0, _EPW)], acc_hbm.at[pl.ds(eb, _EPW)])
    pltpu.sync_copy(dp, degp_hbm.at[wid])


@functools.cache
def _deg_call():
    mesh = plsc.VectorSubcoreMesh(core_axis_name="c", subcore_axis_name="s")
    return pl.kernel(
        _deg_body,
        out_type=(
            jax.ShapeDtypeStruct((_NW, _NACC), jnp.float32),
            jax.ShapeDtypeStruct((_E,), jnp.int32),
            jax.ShapeDtypeStruct((_E,), jnp.int32),
            jax.ShapeDtypeStruct((_NW, 16), jnp.int32),
        ),
        mesh=mesh,
        scratch_types=[
            pltpu.VMEM((_DEG_CH,), jnp.int32),
            pltpu.VMEM((_DEG_CH,), jnp.int32),
            pltpu.VMEM((_NACC,), jnp.float32),
            pltpu.VMEM((_EPW + 48,), jnp.int32),
            pltpu.VMEM((_EPW + 48,), jnp.int32),
            pltpu.VMEM((16,), jnp.int32),
        ],
        compiler_params=pltpu.CompilerParams(needs_layout_passes=False),
    )


def _adj_body(up_hbm, colx_hbm, adst_hbm, out_hbm,
              gi0, gi1, si0, si1, rows0, rows1, zb_v, acc,
              semi0, semi1, semg0, semg1):
    c = lax.axis_index("c")
    s = lax.axis_index("s")
    wid = s * _NC + c
    _zero_buf(zb_v, _ZR)
    _init_acc(acc, zb_v, s)
    plsc.subcore_barrier()
    eb = wid * _EPW
    slots = ((gi0, si0, rows0, semi0, semg0), (gi1, si1, rows1, semi1, semg1))

    def issue_idx(k, gi, si, semi):
        off = pl.multiple_of(eb + k * _CBA, 8)
        pltpu.async_copy(colx_hbm.at[pl.ds(off, _CBA)], gi, semi)
        pltpu.async_copy(adst_hbm.at[pl.ds(off, _CBA)], si, semi)

    def drain_idx(gi, si, semi):
        pltpu.make_async_copy(colx_hbm.at[pl.ds(0, _CBA)], gi, semi).wait()
        pltpu.make_async_copy(adst_hbm.at[pl.ds(0, _CBA)], si, semi).wait()

    issue_idx(0, gi0, si0, semi0)
    drain_idx(gi0, si0, semi0)
    pltpu.async_copy(up_hbm.at[gi0], rows0, semg0)

    def body2(k2, _):
        for b in range(2):
            kk = k2 * 2 + b
            gi, si, rows, semi, semg = slots[b]
            gip, sip, rowsp, semip, semgp = slots[1 - b]

            @pl.when(kk < _NCHA)
            def _():
                @pl.when(kk + 1 < _NCHA)
                def _():
                    issue_idx(kk + 1, gip, sip, semip)

                pltpu.make_async_copy(up_hbm.at[pl.ds(0, _CBA)], rows, semg).wait()

                @pl.when(kk + 1 < _NCHA)
                def _():
                    drain_idx(gip, sip, semip)
                    pltpu.async_copy(up_hbm.at[gip], rowsp, semgp)

                pltpu.sync_copy(rows, acc.at[si], add=True)

        return 0

    lax.fori_loop(0, (_NCHA + 1) // 2, body2, 0)
    plsc.subcore_barrier()
    _write_acc(acc, out_hbm, c, s)


@functools.cache
def _adj_call():
    mesh = plsc.VectorSubcoreMesh(core_axis_name="c", subcore_axis_name="s")
    return pl.kernel(
        _adj_body,
        out_type=jax.ShapeDtypeStruct((_NC, _NACC, _D), jnp.float32),
        mesh=mesh,
        scratch_types=[
            pltpu.VMEM((_CBA,), jnp.int32),
            pltpu.VMEM((_CBA,), jnp.int32),
            pltpu.VMEM((_CBA,), jnp.int32),
            pltpu.VMEM((_CBA,), jnp.int32),
            pltpu.VMEM((_CBA, _D), jnp.float32),
            pltpu.VMEM((_CBA, _D), jnp.float32),
            pltpu.VMEM((_ZR, _D), jnp.float32),
            pltpu.VMEM_SHARED((_NACC, _D), jnp.float32),
            pltpu.SemaphoreType.DMA,
            pltpu.SemaphoreType.DMA,
            pltpu.SemaphoreType.DMA,
            pltpu.SemaphoreType.DMA,
        ],
        compiler_params=pltpu.CompilerParams(needs_layout_passes=False),
    )


def _zupd_body(wp_hbm, arowc_hbm, acolc_hbm, cnt_hbm, zin_hbm, zout_hbm, pt_hbm,
               ia0, ia1, ib0, ib1, wr0, wr1, wc0, wc1, zv0, zv1, zn,
               zb_v, cv16, acc, semi0, semi1, semg0, semg1):
    c = lax.axis_index("c")
    s = lax.axis_index("s")
    wid = s * _NC + c
    pltpu.sync_copy(cnt_hbm.at[wid], cv16)
    nch = jnp.max(cv16[...])
    _zero_buf(zb_v, _ZR)
    _init_acc(acc, zb_v, s)
    plsc.subcore_barrier()
    eb = wid * _EPW
    one = jnp.full((16,), 1.0, jnp.float32)
    slots = (
        (ia0, ib0, wr0, wc0, zv0, semi0, semg0),
        (ia1, ib1, wr1, wc1, zv1, semi1, semg1),
    )

    def issue_idx(k, ia, ib, semi):
        off = pl.multiple_of(eb + k * _CBZ, 8)
        pltpu.async_copy(arowc_hbm.at[pl.ds(off, _CBZ)], ia, semi)
        pltpu.async_copy(acolc_hbm.at[pl.ds(off, _CBZ)], ib, semi)

    def drain_idx(ia, ib, semi):
        pltpu.make_async_copy(arowc_hbm.at[pl.ds(0, _CBZ)], ia, semi).wait()
        pltpu.make_async_copy(acolc_hbm.at[pl.ds(0, _CBZ)], ib, semi).wait()

    def issue_g(k, ia, ib, wr, wc, zv, semg):
        off = pl.multiple_of(eb + k * _CBZ, 8)
        pltpu.async_copy(wp_hbm.at[ia], wr, semg)
        pltpu.async_copy(wp_hbm.at[ib], wc, semg)
        pltpu.async_copy(zin_hbm.at[pl.ds(off, _CBZ)], zv, semg)

    def drain_g(wr, wc, zv, semg):
        pltpu.make_async_copy(wp_hbm.at[pl.ds(0, _CBZ)], wr, semg).wait()
        pltpu.make_async_copy(wp_hbm.at[pl.ds(0, _CBZ)], wc, semg).wait()
        pltpu.make_async_copy(zin_hbm.at[pl.ds(0, _CBZ)], zv, semg).wait()

    issue_idx(0, ia0, ib0, semi0)
    drain_idx(ia0, ib0, semi0)
    issue_g(0, ia0, ib0, wr0, wc0, zv0, semg0)

    def body2(k2, _):
        for b in range(2):
            kk = k2 * 2 + b
            ia, ib, wr, wc, zv, semi, semg = slots[b]
            iap, ibp, wrp, wcp, zvp, semip, semgp = slots[1 - b]

            @pl.when(kk < nch)
            def _():
                @pl.when(kk + 1 < nch)
                def _():
                    issue_idx(kk + 1, iap, ibp, semip)

                drain_g(wr, wc, zv, semg)

                def rows(r2, _2):
                    for u in range(2):
                        r = r2 * 2 + u
                        zcur = []
                        ss = None
                        for j in range(_D // 16):
                            sl = pl.ds(j * 16, 16)
                            v = zv[r, sl] + 2.0 * (wr[r, sl] - wc[r, sl])
                            zcur.append(v)
                            ss = v * v if ss is None else ss + v * v
                        tot = _lane_sum(ss)
                        scale = jnp.minimum(one, _LAM * _rsqrt_nr(tot))
                        for j in range(_D // 16):
                            sl = pl.ds(j * 16, 16)
                            v = scale * zcur[j]
                            zv[r, sl] = v
                            zn[r, sl] = -v
                    return 0

                lax.fori_loop(0, _CBZ // 2, rows, 0)

                @pl.when(kk + 1 < nch)
                def _():
                    drain_idx(iap, ibp, semip)
                    issue_g(kk + 1, iap, ibp, wrp, wcp, zvp, semgp)

                off = pl.multiple_of(eb + kk * _CBZ, 8)
                pltpu.sync_copy(zv, zout_hbm.at[pl.ds(off, _CBZ)])
                pltpu.sync_copy(zv, acc.at[ia], add=True)
                pltpu.sync_copy(zn, acc.at[ib], add=True)

        return 0

    lax.fori_loop(0, (nch + 1) // 2, body2, 0)
    plsc.subcore_barrier()
    _write_acc(acc, pt_hbm, c, s)


@functools.cache
def _zupd_call():
    mesh = plsc.VectorSubcoreMesh(core_axis_name="c", subcore_axis_name="s")
    return pl.kernel(
        _zupd_body,
        out_type=(
            jax.ShapeDtypeStruct((_E, _D), jnp.float32),
            jax.ShapeDtypeStruct((_NC, _NACC, _D), jnp.float32),
        ),
        mesh=mesh,
        scratch_types=[
            pltpu.VMEM((_CBZ,), jnp.int32),
            pltpu.VMEM((_CBZ,), jnp.int32),
            pltpu.VMEM((_CBZ,), jnp.int32),
            pltpu.VMEM((_CBZ,), jnp.int32),
            pltpu.VMEM((_CBZ, _D), jnp.float32),
            pltpu.VMEM((_CBZ, _D), jnp.float32),
            pltpu.VMEM((_CBZ, _D), jnp.float32),
            pltpu.VMEM((_CBZ, _D), jnp.float32),
            pltpu.VMEM((_CBZ, _D), jnp.float32),
            pltpu.VMEM((_CBZ, _D), jnp.float32),
            pltpu.VMEM((_CBZ, _D), jnp.float32),
            pltpu.VMEM((_ZR, _D), jnp.float32),
            pltpu.VMEM((16,), jnp.int32),
            pltpu.VMEM_SHARED((_NACC, _D), jnp.float32),
            pltpu.SemaphoreType.DMA,
            pltpu.SemaphoreType.DMA,
            pltpu.SemaphoreType.DMA,
            pltpu.SemaphoreType.DMA,
        ],
        compiler_params=pltpu.CompilerParams(needs_layout_passes=False),
    )


def _tc_b(xp, xcp, t, p0, p1, dis_col):
    def body(xp_r, xc_r, t_r, p0_r, p1_r, dis_r, y_r, w_r):
        dis = dis_r[...]
        adj = dis * (p0_r[...] + p1_r[...]) + (dis * dis) * xc_r[...]
        y = 0.25 * xp_r[...] + 0.75 * adj
        y_r[...] = y
        w_r[...] = dis * (y - 0.25 * t_r[...])

    return pl.pallas_call(
        body,
        out_shape=[jax.ShapeDtypeStruct((_NACC, _D), jnp.float32)] * 2,
    )(xp, xcp, t, p0, p1, dis_col)


def _tc_d(y, s0, s1, dis_col):
    def body(y_r, s0_r, s1_r, dis_r, t_r, xc_r, up_r):
        dis = dis_r[...]
        t = dis * (s0_r[...] + s1_r[...])
        xc = y_r[...] - 0.25 * t
        t_r[...] = t
        xc_r[...] = xc
        up_r[...] = dis * xc

    return pl.pallas_call(
        body,
        out_shape=[jax.ShapeDtypeStruct((_NACC, _D), jnp.float32)] * 3,
    )(y, s0, s1, dis_col)


def kernel(x, edge_index):
    row = edge_index[0].astype(jnp.int32)
    col = edge_index[1].astype(jnp.int32)

    e_ar = jnp.arange(_E, dtype=jnp.int32)
    spread1 = _N + (e_ar % _PAD)
    adst = jnp.where(row != col, row, spread1)

    degp, arowc, acolc, cnt = _deg_call()(row, col)
    deg = 1.0 + jnp.sum(degp, axis=0)
    ii = jnp.arange(_NACC)
    dis = jnp.where(ii < _N, lax.rsqrt(deg), 0.0).astype(jnp.float32)
    dis_col = dis[:, None]

    xp = jnp.zeros((_NACC, _D), jnp.float32).at[:_N].set(x)
    zacc = jnp.zeros((_NACC, _D), jnp.float32)
    t, xcp, up = _tc_d(xp, zacc, zacc, dis_col)
    z = jnp.zeros((_E, _D), jnp.float32)

    for _ in range(_K):
        p = _adj_call()(up, col, adst)
        y, w = _tc_b(xp, xcp, t, p[0], p[1], dis_col)
        z, sp = _zupd_call()(w, arowc, acolc, cnt, z)
        t, xcp, up = _tc_d(y, sp[0], sp[1], dis_col)
    return xcp[:_N]
